# baseline (speedup 1.0000x reference)
import functools

import jax
import jax.numpy as jnp
from jax import lax
from jax.experimental import pallas as pl
from jax.experimental.pallas import tpu as pltpu
from jax.experimental.pallas import tpu_sc as plsc


def kernel(tokens, W_pos):
    B, P = tokens.shape
    D = W_pos.shape[1]

    info = plsc.get_sparse_core_info()
    NW = info.num_cores * info.num_subcores
    RPW = P // NW
    CH = 8
    NCH = RPW // CH
    NB = 6

    mesh = plsc.VectorSubcoreMesh(core_axis_name="c", subcore_axis_name="s")

    @functools.partial(
        pl.kernel,
        mesh=mesh,
        out_type=jax.ShapeDtypeStruct((B, P, D), W_pos.dtype),
        scratch_types=[
            pltpu.VMEM((NB, CH, D), W_pos.dtype),
            pltpu.SemaphoreType.DMA((NB,)),
            pltpu.SemaphoreType.DMA((NB, B)),
        ],
    )
    def sc_broadcast(w_hbm, out_hbm, buf, in_sem, out_sem):
        wid = lax.axis_index("s") * info.num_cores + lax.axis_index("c")
        base = wid * RPW

        def in_copy(c):
            return pltpu.make_async_copy(
                w_hbm.at[pl.ds(base + c * CH, CH), :],
                buf.at[c % NB],
                in_sem.at[c % NB])

        def out_copy(c, b):
            return pltpu.make_async_copy(
                buf.at[c % NB],
                out_hbm.at[b, pl.ds(base + c * CH, CH), :],
                out_sem.at[c % NB, b])

        in_copy(0).start()
        for c in range(NCH):
            if c + 1 < NCH:
                if c + 1 >= NB:
                    for b in range(B):
                        out_copy(c + 1 - NB, b).wait()
                in_copy(c + 1).start()
            in_copy(c).wait()
            for b in range(B):
                out_copy(c, b).start()
        for c in range(max(0, NCH - NB), NCH):
            for b in range(B):
                out_copy(c, b).wait()

    return sc_broadcast(W_pos)


# speedup vs baseline: 1.6085x; 1.6085x over previous
import jax
import jax.numpy as jnp
from jax.experimental import pallas as pl
from jax.experimental.pallas import tpu as pltpu


def kernel(tokens, W_pos):
    B, P = tokens.shape
    D = W_pos.shape[1]
    CHUNK = 1024
    NC = P // CHUNK
    NH = 2
    HC = CHUNK // NH

    def body(w_hbm, o_hbm, buf, in_sem, out_sem):
        def in_copy(i):
            return pltpu.make_async_copy(
                w_hbm.at[pl.ds(i * CHUNK, CHUNK), :],
                buf.at[pl.ds(i * CHUNK, CHUNK), :],
                in_sem.at[i])

        def out_copy(i, b, h):
            return pltpu.make_async_copy(
                buf.at[pl.ds(i * CHUNK + h * HC, HC), :],
                o_hbm.at[b, pl.ds(i * CHUNK + h * HC, HC), :],
                out_sem.at[i, b, h])

        for i in range(NC):
            in_copy(i).start()
        for i in range(NC):
            in_copy(i).wait()
            for b in range(B):
                for h in range(NH):
                    out_copy(i, b, h).start()
        for i in range(NC):
            for b in range(B):
                for h in range(NH):
                    out_copy(i, b, h).wait()

    out = pl.pallas_call(
        body,
        in_specs=[pl.BlockSpec(memory_space=pl.ANY)],
        out_specs=pl.BlockSpec(memory_space=pl.ANY),
        out_shape=jax.ShapeDtypeStruct((B, P, D), W_pos.dtype),
        scratch_shapes=[
            pltpu.VMEM((P, D), W_pos.dtype),
            pltpu.SemaphoreType.DMA((NC,)),
            pltpu.SemaphoreType.DMA((NC, B, NH)),
        ],
    )(W_pos)
    return out
